# baseline (speedup 1.0000x reference)
import functools

import jax
import jax.numpy as jnp
from jax import lax
from jax.experimental import pallas as pl
from jax.experimental.pallas import tpu as pltpu
from jax.experimental.pallas import tpu_sc as plsc

BATCH = 16384
EMBED_DIM = 128
LANES = 16
NUM_CORES = 2
NUM_SUBCORES = 16
NUM_WORKERS = NUM_CORES * NUM_SUBCORES
B_PER_W = BATCH // NUM_WORKERS
CHUNKS = ((0, 128), (128, 128), (256, 256))
D_CHUNKS = EMBED_DIM // LANES


def _build():
    mesh = plsc.VectorSubcoreMesh(core_axis_name="c", subcore_axis_name="s")

    @functools.partial(
        pl.kernel,
        mesh=mesh,
        out_type=jax.ShapeDtypeStruct((BATCH * EMBED_DIM,), jnp.float32),
        scratch_types=[
            pltpu.VMEM((B_PER_W,), jnp.int32),
            pltpu.VMEM((3, EMBED_DIM), jnp.float32),
            pltpu.VMEM((B_PER_W * EMBED_DIM,), jnp.float32),
            pltpu.SemaphoreType.DMA,
        ],
    )
    def lookup_kernel(idx_hbm, table_hbm, out_hbm, idx_v, table_v, rows_v, sem):
        wid = lax.axis_index("s") * NUM_CORES + lax.axis_index("c")
        base = wid * B_PER_W
        stage = [
            pltpu.async_copy(idx_hbm.at[pl.ds(base, B_PER_W)], idx_v, sem),
            pltpu.async_copy(table_hbm, table_v, sem),
        ]
        for c in stage:
            c.wait()

        trow = [[table_v[r, pl.ds(c * LANES, LANES)] for c in range(D_CHUNKS)]
                for r in range(3)]

        copies = []
        for row0, nrows in CHUNKS:
            @plsc.parallel_loop(0, nrows // LANES, unroll=2)
            def group_body(g, _row0=row0):
                idx16 = idx_v[pl.ds(_row0 + g * LANES, LANES)]
                for k in range(LANES):
                    s = idx16[k]
                    is0 = s == 0
                    is1 = s == 1
                    row_off = (_row0 + g * LANES + k) * EMBED_DIM
                    for c in range(D_CHUNKS):
                        val = jnp.where(is0, trow[0][c],
                                        jnp.where(is1, trow[1][c], trow[2][c]))
                        rows_v[pl.ds(row_off + c * LANES, LANES)] = val
            copies.append(pltpu.async_copy(
                rows_v.at[pl.ds(row0 * EMBED_DIM, nrows * EMBED_DIM)],
                out_hbm.at[pl.ds((base + row0) * EMBED_DIM,
                                 nrows * EMBED_DIM)],
                sem,
            ))
        for c in copies:
            c.wait()

    return lookup_kernel


_sc_lookup = jax.jit(_build())


def kernel(speakers, table):
    out_flat = _sc_lookup(speakers, table)
    return out_flat.reshape(BATCH, EMBED_DIM)


# speedup vs baseline: 1.0243x; 1.0243x over previous
import functools

import jax
import jax.numpy as jnp
from jax import lax
from jax.experimental import pallas as pl
from jax.experimental.pallas import tpu as pltpu
from jax.experimental.pallas import tpu_sc as plsc

BATCH = 16384
EMBED_DIM = 128
LANES = 16
NUM_CORES = 2
NUM_SUBCORES = 16
NUM_WORKERS = NUM_CORES * NUM_SUBCORES
B_PER_W = BATCH // NUM_WORKERS
CHUNKS = ((0, 256), (256, 256))
D_CHUNKS = EMBED_DIM // LANES


def _build():
    mesh = plsc.VectorSubcoreMesh(core_axis_name="c", subcore_axis_name="s")

    @functools.partial(
        pl.kernel,
        mesh=mesh,
        out_type=jax.ShapeDtypeStruct((BATCH * EMBED_DIM,), jnp.float32),
        scratch_types=[
            pltpu.VMEM((B_PER_W,), jnp.int32),
            pltpu.VMEM((3, EMBED_DIM), jnp.float32),
            pltpu.VMEM((B_PER_W * EMBED_DIM,), jnp.float32),
            pltpu.SemaphoreType.DMA,
        ],
    )
    def lookup_kernel(idx_hbm, table_hbm, out_hbm, idx_v, table_v, rows_v, sem):
        wid = lax.axis_index("s") * NUM_CORES + lax.axis_index("c")
        base = wid * B_PER_W
        stage = [
            pltpu.async_copy(idx_hbm.at[pl.ds(base, B_PER_W)], idx_v, sem),
            pltpu.async_copy(table_hbm, table_v, sem),
        ]
        for c in stage:
            c.wait()

        trow = [[table_v[r, pl.ds(c * LANES, LANES)] for c in range(D_CHUNKS)]
                for r in range(3)]

        copies = []
        for row0, nrows in CHUNKS:
            @plsc.parallel_loop(0, nrows // LANES, unroll=2)
            def group_body(g, _row0=row0):
                idx16 = idx_v[pl.ds(_row0 + g * LANES, LANES)]
                for k in range(LANES):
                    s = idx16[k]
                    is0 = s == 0
                    is1 = s == 1
                    row_off = (_row0 + g * LANES + k) * EMBED_DIM
                    for c in range(D_CHUNKS):
                        val = jnp.where(is0, trow[0][c],
                                        jnp.where(is1, trow[1][c], trow[2][c]))
                        rows_v[pl.ds(row_off + c * LANES, LANES)] = val
            copies.append(pltpu.async_copy(
                rows_v.at[pl.ds(row0 * EMBED_DIM, nrows * EMBED_DIM)],
                out_hbm.at[pl.ds((base + row0) * EMBED_DIM,
                                 nrows * EMBED_DIM)],
                sem,
            ))
        for c in copies:
            c.wait()

    return lookup_kernel


_sc_lookup = jax.jit(_build())


def kernel(speakers, table):
    out_flat = _sc_lookup(speakers, table)
    return out_flat.reshape(BATCH, EMBED_DIM)
